# Initial kernel scaffold; baseline (speedup 1.0000x reference)
import jax
import jax.numpy as jnp
from jax.experimental import pallas as pl


def kernel(input, mask):
    raise NotImplementedError("write your pallas kernel here")


# speedup vs baseline: 10.4032x; 10.4032x over previous
import jax
import jax.numpy as jnp
from jax import lax
from jax.experimental import pallas as pl
from jax.experimental.pallas import tpu as pltpu
from jax.experimental.pallas import tpu_sc as plsc

_STRIDE = 2
_B, _H, _W, _C = 8, 112, 112, 96
_N = _H * _W * _C
_M = _N * _STRIDE * _STRIDE
_NC, _NS, _L = 2, 16, 16
_NQ = 4
_CH = _M // _NQ
_TRASH = 256
_P = _N // _NS
_NSTEP = 7
_K = _P // _NSTEP
_STRIPE = _CH // _NS
_NZ = 7
_ZB = _STRIPE // _NZ
_BPC = _B // _NC


def _unpool_body(x_hbm, m_hbm, out_hbm, mask_v, vals_v, idx_v, zero_v, acc_sh):
    cid = lax.axis_index("c")
    sid = lax.axis_index("s")

    z16 = jnp.zeros((_L,), jnp.float32)

    @pl.loop(0, _ZB // _L)
    def _(i):
        zero_v[pl.ds(i * _L, _L)] = z16

    trash_vec = jnp.full((_L,), _CH, jnp.uint32) + lax.iota(jnp.uint32, _L)

    @pl.loop(0, _BPC)
    def _(b_loc):
        b = cid * _BPC + b_loc

        @pl.loop(0, _NQ)
        def _(q):
            base = q * _CH

            @pl.loop(0, _NZ)
            def _(z):
                pltpu.sync_copy(
                    zero_v, acc_sh.at[pl.ds(sid * _STRIPE + z * _ZB, _ZB)]
                )

            plsc.subcore_barrier()

            base_vec = jnp.full((_L,), base, jnp.int32)

            @pl.loop(0, _NSTEP)
            def _(st):
                off = b * _N + sid * _P + st * _K
                pltpu.sync_copy(m_hbm.at[pl.ds(off, _K)], mask_v)
                pltpu.sync_copy(x_hbm.at[pl.ds(off, _K)], vals_v)

                @plsc.parallel_loop(0, _K // _L, unroll=4)
                def _(i):
                    m16 = mask_v[pl.ds(i * _L, _L)]
                    rel = plsc.bitcast(m16 - base_vec, jnp.uint32)
                    idx = jnp.minimum(rel, trash_vec)
                    idx_v[pl.ds(i * _L, _L)] = plsc.bitcast(idx, jnp.int32)

                pltpu.sync_copy(vals_v, acc_sh.at[idx_v], add=True)

            plsc.subcore_barrier()

            out_off = b * _M + base + sid * _STRIPE
            pltpu.sync_copy(
                acc_sh.at[pl.ds(sid * _STRIPE, _STRIPE)],
                out_hbm.at[pl.ds(out_off, _STRIPE)],
            )


@jax.jit
def kernel(input, mask):
    x = input.reshape(-1)
    m = mask.reshape(-1)
    mesh = plsc.VectorSubcoreMesh(core_axis_name="c", subcore_axis_name="s")
    out = pl.kernel(
        _unpool_body,
        out_type=jax.ShapeDtypeStruct((_B * _M,), jnp.float32),
        mesh=mesh,
        scratch_types=[
            pltpu.VMEM((_K,), jnp.int32),
            pltpu.VMEM((_K,), jnp.float32),
            pltpu.VMEM((_K,), jnp.int32),
            pltpu.VMEM((_ZB,), jnp.float32),
            pltpu.VMEM_SHARED((_CH + _TRASH,), jnp.float32),
        ],
    )(x, m)
    return out.reshape(_B, _H * _STRIDE, _W * _STRIDE, _C)


# speedup vs baseline: 22.3399x; 2.1474x over previous
import jax
import jax.numpy as jnp
from jax import lax
from jax.experimental import pallas as pl
from jax.experimental.pallas import tpu as pltpu
from jax.experimental.pallas import tpu_sc as plsc

_STRIDE = 2
_B, _H, _W, _C = 8, 112, 112, 96
_N = _H * _W * _C
_M = _N * _STRIDE * _STRIDE
_NC, _NS, _L = 2, 16, 16
_NQ = 4
_CH = _M // _NQ
_TRASH = 4096
_P = _N // _NS
_NSTEP = 7
_K = _P // _NSTEP
_STRIPE = _CH // _NS
_NZ = 7
_ZB = _STRIPE // _NZ
_BPC = _B // _NC


def _unpool_body(x_hbm, m_hbm, out_hbm, mask_v, vals_v, idx_v, zero_v, acc_sh):
    cid = lax.axis_index("c")
    sid = lax.axis_index("s")

    z16 = jnp.zeros((_L,), jnp.float32)

    @pl.loop(0, _ZB // _L)
    def _(i):
        zero_v[pl.ds(i * _L, _L)] = z16

    ch_vec = jnp.full((_L,), _CH, jnp.uint32)
    tmask_vec = jnp.full((_L,), _TRASH - 1, jnp.uint32)

    @pl.loop(0, _BPC)
    def _(b_loc):
        b = cid * _BPC + b_loc

        @pl.loop(0, _NQ)
        def _(q):
            base = q * _CH

            @pl.loop(0, _NZ)
            def _(z):
                pltpu.sync_copy(
                    zero_v, acc_sh.at[pl.ds(sid * _STRIPE + z * _ZB, _ZB)]
                )

            plsc.subcore_barrier()

            base_vec = jnp.full((_L,), base, jnp.int32)

            @pl.loop(0, _NSTEP)
            def _(st):
                off = b * _N + sid * _P + st * _K
                pltpu.sync_copy(m_hbm.at[pl.ds(off, _K)], mask_v)
                pltpu.sync_copy(x_hbm.at[pl.ds(off, _K)], vals_v)

                @plsc.parallel_loop(0, _K // _L, unroll=4)
                def _(i):
                    m16 = mask_v[pl.ds(i * _L, _L)]
                    mu = plsc.bitcast(m16, jnp.uint32)
                    rel = plsc.bitcast(m16 - base_vec, jnp.uint32)
                    trash = ch_vec + (mu & tmask_vec)
                    idx = jnp.minimum(rel, trash)
                    idx_v[pl.ds(i * _L, _L)] = plsc.bitcast(idx, jnp.int32)

                pltpu.sync_copy(vals_v, acc_sh.at[idx_v], add=True)

            plsc.subcore_barrier()

            out_off = b * _M + base + sid * _STRIPE
            pltpu.sync_copy(
                acc_sh.at[pl.ds(sid * _STRIPE, _STRIPE)],
                out_hbm.at[pl.ds(out_off, _STRIPE)],
            )


@jax.jit
def kernel(input, mask):
    x = input.reshape(-1)
    m = mask.reshape(-1)
    mesh = plsc.VectorSubcoreMesh(core_axis_name="c", subcore_axis_name="s")
    out = pl.kernel(
        _unpool_body,
        out_type=jax.ShapeDtypeStruct((_B * _M,), jnp.float32),
        mesh=mesh,
        scratch_types=[
            pltpu.VMEM((_K,), jnp.int32),
            pltpu.VMEM((_K,), jnp.float32),
            pltpu.VMEM((_K,), jnp.int32),
            pltpu.VMEM((_ZB,), jnp.float32),
            pltpu.VMEM_SHARED((_CH + _TRASH,), jnp.float32),
        ],
    )(x, m)
    return out.reshape(_B, _H * _STRIDE, _W * _STRIDE, _C)


# speedup vs baseline: 22.3685x; 1.0013x over previous
import jax
import jax.numpy as jnp
from jax import lax
from jax.experimental import pallas as pl
from jax.experimental.pallas import tpu as pltpu
from jax.experimental.pallas import tpu_sc as plsc

_STRIDE = 2
_B, _H, _W, _C = 8, 112, 112, 96
_N = _H * _W * _C
_M = _N * _STRIDE * _STRIDE
_NC, _NS, _L = 2, 16, 16
_NQ = 4
_CH = _M // _NQ
_TRASH = 131072
_P = _N // _NS
_NSTEP = 7
_K = _P // _NSTEP
_STRIPE = _CH // _NS
_NZ = 7
_ZB = _STRIPE // _NZ
_BPC = _B // _NC


def _unpool_body(x_hbm, m_hbm, out_hbm, mask_v, vals_v, idx_v, zero_v, acc_sh):
    cid = lax.axis_index("c")
    sid = lax.axis_index("s")

    z16 = jnp.zeros((_L,), jnp.float32)

    @pl.loop(0, _ZB // _L)
    def _(i):
        zero_v[pl.ds(i * _L, _L)] = z16

    ch_vec = jnp.full((_L,), _CH, jnp.uint32)
    tmask_vec = jnp.full((_L,), _TRASH - 1, jnp.uint32)

    @pl.loop(0, _BPC)
    def _(b_loc):
        b = cid * _BPC + b_loc

        @pl.loop(0, _NQ)
        def _(q):
            base = q * _CH

            @pl.loop(0, _NZ)
            def _(z):
                pltpu.sync_copy(
                    zero_v, acc_sh.at[pl.ds(sid * _STRIPE + z * _ZB, _ZB)]
                )

            plsc.subcore_barrier()

            base_vec = jnp.full((_L,), base, jnp.int32)

            @pl.loop(0, _NSTEP)
            def _(st):
                off = b * _N + sid * _P + st * _K
                pltpu.sync_copy(m_hbm.at[pl.ds(off, _K)], mask_v)
                pltpu.sync_copy(x_hbm.at[pl.ds(off, _K)], vals_v)

                @plsc.parallel_loop(0, _K // _L, unroll=4)
                def _(i):
                    m16 = mask_v[pl.ds(i * _L, _L)]
                    mu = plsc.bitcast(m16, jnp.uint32)
                    rel = plsc.bitcast(m16 - base_vec, jnp.uint32)
                    trash = ch_vec + (mu & tmask_vec)
                    idx = jnp.minimum(rel, trash)
                    idx_v[pl.ds(i * _L, _L)] = plsc.bitcast(idx, jnp.int32)

                pltpu.sync_copy(vals_v, acc_sh.at[idx_v], add=True)

            plsc.subcore_barrier()

            out_off = b * _M + base + sid * _STRIPE
            pltpu.sync_copy(
                acc_sh.at[pl.ds(sid * _STRIPE, _STRIPE)],
                out_hbm.at[pl.ds(out_off, _STRIPE)],
            )


@jax.jit
def kernel(input, mask):
    x = input.reshape(-1)
    m = mask.reshape(-1)
    mesh = plsc.VectorSubcoreMesh(core_axis_name="c", subcore_axis_name="s")
    out = pl.kernel(
        _unpool_body,
        out_type=jax.ShapeDtypeStruct((_B * _M,), jnp.float32),
        mesh=mesh,
        scratch_types=[
            pltpu.VMEM((_K,), jnp.int32),
            pltpu.VMEM((_K,), jnp.float32),
            pltpu.VMEM((_K,), jnp.int32),
            pltpu.VMEM((_ZB,), jnp.float32),
            pltpu.VMEM_SHARED((_CH + _TRASH,), jnp.float32),
        ],
    )(x, m)
    return out.reshape(_B, _H * _STRIDE, _W * _STRIDE, _C)
